# baseline (speedup 1.0000x reference)
import functools

import jax
import jax.numpy as jnp
from jax import lax
from jax.experimental import pallas as pl
from jax.experimental.pallas import tpu as pltpu
from jax.experimental.pallas import tpu_sc as plsc

_B = 16384
_D = 128
_Z = 512

_NSPLIT = 4
_BC = _B // _NSPLIT

_NC = 2
_NS = 16
_NW = _NC * _NS
_BPW = _BC // _NW
_CH = 128
_NCH = _BPW // _CH


def _make_sc_gather(chunk):
    mesh = plsc.VectorSubcoreMesh(core_axis_name="c", subcore_axis_name="s")

    @functools.partial(
        pl.kernel,
        mesh=mesh,
        out_type=jax.ShapeDtypeStruct((_BC, _D), jnp.float32),
        scratch_types=[
            pltpu.VMEM((_NCH, _CH), jnp.int32),
            pltpu.VMEM((_BPW, _D), jnp.float32),
            pltpu.SemaphoreType.DMA,
        ],
    )
    def gather_kernel(table_hbm, idx_hbm, out_hbm, idx_v, rows_v, sem):
        wid = lax.axis_index("s") * _NC + lax.axis_index("c")
        pltpu.sync_copy(idx_hbm.at[chunk * _NW + wid], idx_v)
        cps = [
            pltpu.async_copy(
                table_hbm.at[idx_v.at[j]],
                rows_v.at[pl.ds(j * _CH, _CH)],
                sem,
            )
            for j in range(_NCH)
        ]
        for cp in cps:
            cp.wait()
        pltpu.sync_copy(rows_v, out_hbm.at[pl.ds(wid * _BPW, _BPW)])

    return gather_kernel


_sc_gathers = [_make_sc_gather(k) for k in range(_NSPLIT)]

_BT = 1024


def _tc_body(out_alias_ref, rows_ref, modes_ref, out_ref):
    del out_alias_ref
    x = rows_ref[...]
    m = jnp.max(x, axis=-1, keepdims=True)
    e = jnp.exp(x - m)
    z = jnp.dot(e, modes_ref[...], preferred_element_type=jnp.float32)
    ss = jnp.maximum(jnp.sum(z * z, axis=-1, keepdims=True), 1e-24)
    out_ref[...] = (z * lax.rsqrt(ss))[:, None, :]


def _tc_first_body(rows_ref, modes_ref, out_ref):
    _tc_body(None, rows_ref, modes_ref, out_ref)


def _tc_chunk(out_buf, rows, modes, chunk):
    base = chunk * (_BC // _BT)
    return pl.pallas_call(
        _tc_body,
        grid=(_BC // _BT,),
        in_specs=[
            pl.BlockSpec(memory_space=pl.ANY),
            pl.BlockSpec((_BT, _D), lambda i: (i, 0)),
            pl.BlockSpec((_D, _Z), lambda i: (0, 0)),
        ],
        out_specs=pl.BlockSpec((_BT, 1, _Z), lambda i, b=base: (b + i, 0, 0)),
        out_shape=jax.ShapeDtypeStruct((_B, 1, _Z), jnp.float32),
        input_output_aliases={0: 0},
    )(out_buf, rows, modes)


def _tc_first(rows, modes):
    return pl.pallas_call(
        _tc_first_body,
        grid=(_BC // _BT,),
        in_specs=[
            pl.BlockSpec((_BT, _D), lambda i: (i, 0)),
            pl.BlockSpec((_D, _Z), lambda i: (0, 0)),
        ],
        out_specs=pl.BlockSpec((_BT, 1, _Z), lambda i: (i, 0, 0)),
        out_shape=jax.ShapeDtypeStruct((_B, 1, _Z), jnp.float32),
    )(rows, modes)


@jax.jit
def kernel(idx, weight_embedding, main_modes):
    idx32 = idx.astype(jnp.int32).reshape(_NSPLIT * _NW, _NCH, _CH)
    rows = [_sc_gathers[k](weight_embedding, idx32) for k in range(_NSPLIT)]
    out = _tc_first(rows[0], main_modes)
    for k in range(1, _NSPLIT):
        out = _tc_chunk(out, rows[k], main_modes, k)
    return out


# speedup vs baseline: 1.1154x; 1.1154x over previous
import functools

import jax
import jax.numpy as jnp
from jax import lax
from jax.experimental import pallas as pl
from jax.experimental.pallas import tpu as pltpu
from jax.experimental.pallas import tpu_sc as plsc

_B = 16384
_D = 128
_Z = 512

_NSPLIT = 1
_BC = _B // _NSPLIT

_NC = 2
_NS = 16
_NW = _NC * _NS
_BPW = _BC // _NW
_CH = 128
_NCH = _BPW // _CH


def _make_sc_gather(chunk):
    mesh = plsc.VectorSubcoreMesh(core_axis_name="c", subcore_axis_name="s")

    @functools.partial(
        pl.kernel,
        mesh=mesh,
        out_type=jax.ShapeDtypeStruct((_BC, _D), jnp.float32),
        scratch_types=[
            pltpu.VMEM((_NCH, _CH), jnp.int32),
            pltpu.VMEM((_BPW, _D), jnp.float32),
            pltpu.SemaphoreType.DMA,
        ],
    )
    def gather_kernel(table_hbm, idx_hbm, out_hbm, idx_v, rows_v, sem):
        wid = lax.axis_index("s") * _NC + lax.axis_index("c")
        pltpu.sync_copy(idx_hbm.at[chunk * _NW + wid], idx_v)
        cps = [
            pltpu.async_copy(
                table_hbm.at[idx_v.at[j]],
                rows_v.at[pl.ds(j * _CH, _CH)],
                sem,
            )
            for j in range(_NCH)
        ]
        for cp in cps:
            cp.wait()
        pltpu.sync_copy(rows_v, out_hbm.at[pl.ds(wid * _BPW, _BPW)])

    return gather_kernel


_sc_gathers = [_make_sc_gather(k) for k in range(_NSPLIT)]

_BT = 1024


def _tc_body(out_alias_ref, rows_ref, modes_ref, out_ref):
    del out_alias_ref
    x = rows_ref[...]
    m = jnp.max(x, axis=-1, keepdims=True)
    e = jnp.exp(x - m)
    z = jnp.dot(e, modes_ref[...], preferred_element_type=jnp.float32)
    ss = jnp.maximum(jnp.sum(z * z, axis=-1, keepdims=True), 1e-24)
    out_ref[...] = (z * lax.rsqrt(ss)).reshape(_BT, 4, 128)


def _tc_first_body(rows_ref, modes_ref, out_ref):
    _tc_body(None, rows_ref, modes_ref, out_ref)


def _tc_chunk(out_buf, rows, modes, chunk):
    base = chunk * (_BC // _BT)
    return pl.pallas_call(
        _tc_body,
        grid=(_BC // _BT,),
        in_specs=[
            pl.BlockSpec(memory_space=pl.ANY),
            pl.BlockSpec((_BT, _D), lambda i: (i, 0)),
            pl.BlockSpec((_D, _Z), lambda i: (0, 0)),
        ],
        out_specs=pl.BlockSpec((_BT, 4, 128), lambda i, b=base: (b + i, 0, 0)),
        out_shape=jax.ShapeDtypeStruct((_B, 4, 128), jnp.float32),
        input_output_aliases={0: 0},
    )(out_buf, rows, modes)


def _tc_first(rows, modes):
    return pl.pallas_call(
        _tc_first_body,
        grid=(_BC // _BT,),
        in_specs=[
            pl.BlockSpec((_BT, _D), lambda i: (i, 0)),
            pl.BlockSpec((_D, _Z), lambda i: (0, 0)),
        ],
        out_specs=pl.BlockSpec((_BT, 4, 128), lambda i: (i, 0, 0)),
        out_shape=jax.ShapeDtypeStruct((_B, 4, 128), jnp.float32),
    )(rows, modes)


@jax.jit
def kernel(idx, weight_embedding, main_modes):
    idx32 = idx.astype(jnp.int32).reshape(_NSPLIT * _NW, _NCH, _CH)
    rows = [_sc_gathers[k](weight_embedding, idx32) for k in range(_NSPLIT)]
    out = _tc_first(rows[0], main_modes)
    for k in range(1, _NSPLIT):
        out = _tc_chunk(out, rows[k], main_modes, k)
    return out.reshape(_B, 1, _Z)


# speedup vs baseline: 1.1371x; 1.0194x over previous
import functools

import jax
import jax.numpy as jnp
from jax import lax
from jax.experimental import pallas as pl
from jax.experimental.pallas import tpu as pltpu
from jax.experimental.pallas import tpu_sc as plsc

_B = 16384
_D = 128
_Z = 512

_CHUNKS = (4096, 12288)
_STARTS = (0, 4096)

_NC = 2
_NS = 16
_NW = _NC * _NS
_CH = 128


def _make_sc_gather(start, size):
    bpw = size // _NW
    nch = bpw // _CH
    mesh = plsc.VectorSubcoreMesh(core_axis_name="c", subcore_axis_name="s")

    @functools.partial(
        pl.kernel,
        mesh=mesh,
        out_type=jax.ShapeDtypeStruct((size, _D), jnp.float32),
        scratch_types=[
            pltpu.VMEM((nch, 1, _CH), jnp.int32),
            pltpu.VMEM((bpw, _D), jnp.float32),
            pltpu.SemaphoreType.DMA,
        ],
    )
    def gather_kernel(table_hbm, idx_hbm, out_hbm, idx_v, rows_v, sem):
        wid = lax.axis_index("s") * _NC + lax.axis_index("c")
        pltpu.sync_copy(idx_hbm.at[pl.ds(start // _CH + wid * nch, nch)], idx_v)
        cps = [
            pltpu.async_copy(
                table_hbm.at[idx_v.at[j, 0]],
                rows_v.at[pl.ds(j * _CH, _CH)],
                sem,
            )
            for j in range(nch)
        ]
        for cp in cps:
            cp.wait()
        pltpu.sync_copy(rows_v, out_hbm.at[pl.ds(wid * bpw, bpw)])

    return gather_kernel


_BT = 2048


def _tc_body(out_alias_ref, rows_ref, modes_ref, out_ref):
    del out_alias_ref
    x = rows_ref[...]
    m = jnp.max(x, axis=-1, keepdims=True)
    e = jnp.exp(x - m)
    z = jnp.dot(e, modes_ref[...], preferred_element_type=jnp.float32)
    ss = jnp.maximum(jnp.sum(z * z, axis=-1, keepdims=True), 1e-24)
    out_ref[...] = (z * lax.rsqrt(ss)).reshape(_BT, 4, 128)


def _tc_first_body(rows_ref, modes_ref, out_ref):
    _tc_body(None, rows_ref, modes_ref, out_ref)


def _tc_first(rows, modes, size):
    return pl.pallas_call(
        _tc_first_body,
        grid=(size // _BT,),
        in_specs=[
            pl.BlockSpec((_BT, _D), lambda i: (i, 0)),
            pl.BlockSpec((_D, _Z), lambda i: (0, 0)),
        ],
        out_specs=pl.BlockSpec((_BT, 4, 128), lambda i: (i, 0, 0)),
        out_shape=jax.ShapeDtypeStruct((_B, 4, 128), jnp.float32),
    )(rows, modes)


def _tc_chunk(out_buf, rows, modes, start, size):
    base = start // _BT
    return pl.pallas_call(
        _tc_body,
        grid=(size // _BT,),
        in_specs=[
            pl.BlockSpec(memory_space=pl.ANY),
            pl.BlockSpec((_BT, _D), lambda i: (i, 0)),
            pl.BlockSpec((_D, _Z), lambda i: (0, 0)),
        ],
        out_specs=pl.BlockSpec((_BT, 4, 128), lambda i, b=base: (b + i, 0, 0)),
        out_shape=jax.ShapeDtypeStruct((_B, 4, 128), jnp.float32),
        input_output_aliases={0: 0},
    )(out_buf, rows, modes)


_sc_gathers = [
    _make_sc_gather(s, c) for s, c in zip(_STARTS, _CHUNKS)
]


@jax.jit
def kernel(idx, weight_embedding, main_modes):
    idx32 = idx.astype(jnp.int32).reshape(_B // _CH, 1, _CH)
    rows = [g(weight_embedding, idx32) for g in _sc_gathers]
    out = _tc_first(rows[0], main_modes, _CHUNKS[0])
    out = _tc_chunk(out, rows[1], main_modes, _STARTS[1], _CHUNKS[1])
    return out.reshape(_B, 1, _Z)


# speedup vs baseline: 1.1737x; 1.0322x over previous
import functools

import jax
import jax.numpy as jnp
from jax import lax
from jax.experimental import pallas as pl
from jax.experimental.pallas import tpu as pltpu
from jax.experimental.pallas import tpu_sc as plsc

_B = 16384
_D = 128
_Z = 512

_CHUNKS = (8192, 8192)
_STARTS = (0, 8192)

_NC = 2
_NS = 16
_NW = _NC * _NS
_CH = 128


def _make_sc_gather(start, size):
    bpw = size // _NW
    nch = bpw // _CH
    mesh = plsc.VectorSubcoreMesh(core_axis_name="c", subcore_axis_name="s")

    @functools.partial(
        pl.kernel,
        mesh=mesh,
        out_type=jax.ShapeDtypeStruct((size, _D), jnp.float32),
        scratch_types=[
            pltpu.VMEM((nch, 1, _CH), jnp.int32),
            pltpu.VMEM((bpw, _D), jnp.float32),
            pltpu.SemaphoreType.DMA,
        ],
    )
    def gather_kernel(table_hbm, idx_hbm, out_hbm, idx_v, rows_v, sem):
        wid = lax.axis_index("s") * _NC + lax.axis_index("c")
        pltpu.sync_copy(idx_hbm.at[pl.ds(start // _CH + wid * nch, nch)], idx_v)
        cps = [
            pltpu.async_copy(
                table_hbm.at[idx_v.at[j, 0]],
                rows_v.at[pl.ds(j * _CH, _CH)],
                sem,
            )
            for j in range(nch)
        ]
        for cp in cps:
            cp.wait()
        pltpu.sync_copy(rows_v, out_hbm.at[pl.ds(wid * bpw, bpw)])

    return gather_kernel


_BT = 2048


def _tc_body(out_alias_ref, rows_ref, modes_ref, out_ref):
    del out_alias_ref
    x = rows_ref[...]
    m = jnp.max(x, axis=-1, keepdims=True)
    e = jnp.exp(x - m)
    z = jnp.dot(e, modes_ref[...], preferred_element_type=jnp.float32)
    ss = jnp.maximum(jnp.sum(z * z, axis=-1, keepdims=True), 1e-24)
    out_ref[...] = (z * lax.rsqrt(ss)).reshape(_BT, 4, 128)


def _tc_first_body(rows_ref, modes_ref, out_ref):
    _tc_body(None, rows_ref, modes_ref, out_ref)


def _tc_first(rows, modes, size):
    return pl.pallas_call(
        _tc_first_body,
        grid=(size // _BT,),
        in_specs=[
            pl.BlockSpec((_BT, _D), lambda i: (i, 0)),
            pl.BlockSpec((_D, _Z), lambda i: (0, 0)),
        ],
        out_specs=pl.BlockSpec((_BT, 4, 128), lambda i: (i, 0, 0)),
        out_shape=jax.ShapeDtypeStruct((_B, 4, 128), jnp.float32),
    )(rows, modes)


def _tc_chunk(out_buf, rows, modes, start, size):
    base = start // _BT
    return pl.pallas_call(
        _tc_body,
        grid=(size // _BT,),
        in_specs=[
            pl.BlockSpec(memory_space=pl.ANY),
            pl.BlockSpec((_BT, _D), lambda i: (i, 0)),
            pl.BlockSpec((_D, _Z), lambda i: (0, 0)),
        ],
        out_specs=pl.BlockSpec((_BT, 4, 128), lambda i, b=base: (b + i, 0, 0)),
        out_shape=jax.ShapeDtypeStruct((_B, 4, 128), jnp.float32),
        input_output_aliases={0: 0},
    )(out_buf, rows, modes)


_sc_gathers = [
    _make_sc_gather(s, c) for s, c in zip(_STARTS, _CHUNKS)
]


@jax.jit
def kernel(idx, weight_embedding, main_modes):
    idx32 = idx.astype(jnp.int32).reshape(_B // _CH, 1, _CH)
    rows = [g(weight_embedding, idx32) for g in _sc_gathers]
    out = _tc_first(rows[0], main_modes, _CHUNKS[0])
    out = _tc_chunk(out, rows[1], main_modes, _STARTS[1], _CHUNKS[1])
    return out.reshape(_B, 1, _Z)
